# baseline (speedup 1.0000x reference)
import functools

import jax
import jax.numpy as jnp
from jax import lax
from jax.experimental import pallas as pl
from jax.experimental.pallas import tpu as pltpu
from jax.experimental.pallas import tpu_sc as plsc

D_MODEL = 32
LANES = 16
NBUF = 4
C = 160


@jax.jit
def _gembed(idx_flat, W_mean, W_logstd):
    B = idx_flat.shape[0]
    info = plsc.get_sparse_core_info()
    NC, NS = info.num_cores, info.num_subcores
    NW = NC * NS
    b_per_w = B // NW
    n_chunks = b_per_w // C
    n_grp = n_chunks // NBUF
    CE = C * D_MODEL
    assert b_per_w % (C * NBUF) == 0 and C % 8 == 0

    mesh = plsc.VectorSubcoreMesh(core_axis_name="c", subcore_axis_name="s")

    scratch = (
        [pltpu.VMEM((b_per_w,), jnp.int32)]
        + [pltpu.VMEM((C, D_MODEL), jnp.float32) for _ in range(2 * NBUF)]
        + [pltpu.VMEM((CE,), jnp.float32) for _ in range(2 * NBUF)]
        + [pltpu.SemaphoreType.DMA for _ in range(4 * NBUF)]
    )

    @functools.partial(
        pl.kernel,
        mesh=mesh,
        compiler_params=pltpu.CompilerParams(use_tc_tiling_on_sc=False),
        out_type=[
            jax.ShapeDtypeStruct((B * D_MODEL,), jnp.float32),
            jax.ShapeDtypeStruct((B * D_MODEL,), jnp.float32),
        ],
        scratch_types=scratch,
    )
    def k(idx_hbm, wm_hbm, ws_hbm, mean_hbm, std_hbm, idx_v, *rest):
        gbuf_m = rest[0:NBUF]
        gbuf_s = rest[NBUF:2 * NBUF]
        st_m = rest[2 * NBUF:3 * NBUF]
        st_s = rest[3 * NBUF:4 * NBUF]
        sem_gm = rest[4 * NBUF:5 * NBUF]
        sem_gs = rest[5 * NBUF:6 * NBUF]
        sem_wm = rest[6 * NBUF:7 * NBUF]
        sem_ws = rest[7 * NBUF:8 * NBUF]

        wid = lax.axis_index("s") * NC + lax.axis_index("c")
        base = pl.multiple_of(wid * b_per_w, 8)
        ebase = pl.multiple_of(wid * b_per_w * D_MODEL, 8)
        pltpu.sync_copy(idx_hbm.at[pl.ds(base, b_per_w)], idx_v)

        def start_gathers(kc, b):
            off = pl.multiple_of(kc * C, 8)
            pltpu.async_copy(wm_hbm.at[idx_v.at[pl.ds(off, C)]], gbuf_m[b],
                             sem_gm[b])
            pltpu.async_copy(ws_hbm.at[idx_v.at[pl.ds(off, C)]], gbuf_s[b],
                             sem_gs[b])

        for b in range(NBUF):
            start_gathers(b, b)

        @pl.loop(0, n_grp)
        def grp_loop(grp):
            for b in range(NBUF):
                kc = grp * NBUF + b
                eoff = pl.multiple_of(kc * CE, 8)

                pltpu.make_async_copy(wm_hbm.at[idx_v.at[pl.ds(0, C)]],
                                      gbuf_m[b], sem_gm[b]).wait()

                @pl.when(grp > 0)
                def _():
                    pltpu.make_async_copy(st_m[b],
                                          mean_hbm.at[pl.ds(ebase, CE)],
                                          sem_wm[b]).wait()

                gm, sm = gbuf_m[b], st_m[b]

                @plsc.parallel_loop(0, C, step=4)
                def repack_m(i):
                    for r in range(4):
                        for h in range(D_MODEL // LANES):
                            sm[pl.ds((i + r) * D_MODEL + h * LANES, LANES)] = \
                                gm[i + r, pl.ds(h * LANES, LANES)]

                pltpu.async_copy(st_m[b], mean_hbm.at[pl.ds(ebase + eoff, CE)],
                                 sem_wm[b])

                pltpu.make_async_copy(ws_hbm.at[idx_v.at[pl.ds(0, C)]],
                                      gbuf_s[b], sem_gs[b]).wait()

                @pl.when(grp > 0)
                def _():
                    pltpu.make_async_copy(st_s[b],
                                          std_hbm.at[pl.ds(ebase, CE)],
                                          sem_ws[b]).wait()

                gs, ss = gbuf_s[b], st_s[b]

                @plsc.parallel_loop(0, C, step=4)
                def repack_s(i):
                    for r in range(4):
                        for h in range(D_MODEL // LANES):
                            ss[pl.ds((i + r) * D_MODEL + h * LANES, LANES)] = \
                                jnp.exp(gs[i + r, pl.ds(h * LANES, LANES)])

                pltpu.async_copy(st_s[b], std_hbm.at[pl.ds(ebase + eoff, CE)],
                                 sem_ws[b])

                @pl.when(grp < n_grp - 1)
                def _():
                    start_gathers((grp + 1) * NBUF + b, b)

        for b in range(NBUF):
            pltpu.make_async_copy(st_m[b], mean_hbm.at[pl.ds(ebase, CE)],
                                  sem_wm[b]).wait()
            pltpu.make_async_copy(st_s[b], std_hbm.at[pl.ds(ebase, CE)],
                                  sem_ws[b]).wait()

    return k(idx_flat, W_mean, W_logstd)


def kernel(idx, W_mean, W_logstd):
    B0, H = idx.shape
    idx_flat = idx.reshape(B0 * H).astype(jnp.int32)
    mean_flat, std_flat = _gembed(idx_flat, W_mean, W_logstd)
    return (mean_flat.reshape(B0, H, D_MODEL), std_flat.reshape(B0, H, D_MODEL))
